# baseline (speedup 1.0000x reference)
import functools

import jax
import jax.numpy as jnp
from jax import lax
from jax.experimental import pallas as pl
from jax.experimental.pallas import tpu as pltpu
from jax.experimental.pallas import tpu_sc as plsc

_SELU_SCALE = 1.0507009873554805
_SELU_ALPHA = 1.6732632423543772

_NC = 2
_NS = 16
_CH = 128
_D = 8
_CORE0_SHARE = 0.5


def _selu(x):
    return _SELU_SCALE * jnp.where(x > 0, x, _SELU_ALPHA * (jnp.exp(x) - 1.0))


def _matmul(x, w):
    n, d_in = x.shape
    d_out = w.shape[1]
    rb = 2000 if n % 2000 == 0 else 8
    grid = n // rb

    def body(x_ref, w_ref, o_ref):
        o_ref[:] = jnp.dot(x_ref[:], w_ref[:], preferred_element_type=jnp.float32)

    return pl.pallas_call(
        body,
        grid=(grid,),
        in_specs=[
            pl.BlockSpec((rb, d_in), lambda i: (i, 0)),
            pl.BlockSpec((d_in, d_out), lambda i: (0, 0)),
        ],
        out_specs=pl.BlockSpec((rb, d_out), lambda i: (i, 0)),
        out_shape=jax.ShapeDtypeStruct((n, d_out), jnp.float32),
    )(x, w)


def _spmm_sc(h, edges3, zeros_blk, acc_rows, n_chunks):
    n, d_h = h.shape
    zr = acc_rows // _NS

    n_groups = n_chunks // _D
    p0 = max(1, min(n_groups // _NS - 1, round(n_groups * _CORE0_SHARE / _NS)))
    rest = n_groups - p0 * _NS
    p1 = rest // _NS
    extra = rest - p1 * _NS
    nc0 = _D * p0
    nc1 = _D * p1
    nc_max = _D * max(p0, p1 + (1 if extra else 0))

    mesh = plsc.VectorSubcoreMesh(
        core_axis_name="c", subcore_axis_name="s",
        num_cores=_NC, num_subcores=_NS)

    @functools.partial(
        pl.kernel,
        out_type=jax.ShapeDtypeStruct((acc_rows, _NC * d_h), jnp.float32),
        mesh=mesh,
        scratch_types=[
            pltpu.VMEM((nc_max, _CH), jnp.int32),
            pltpu.VMEM((nc_max, _CH), jnp.int32),
        ] + [pltpu.VMEM((_CH, d_h), jnp.float32)] * _D + [
            pltpu.VMEM_SHARED((acc_rows, d_h), jnp.float32),
        ] + [pltpu.SemaphoreType.DMA] * _D,
        compiler_params=pltpu.CompilerParams(use_tc_tiling_on_sc=False),
    )
    def spmm(h_hbm, edges_hbm, zeros_hbm, out_hbm,
             src_v, dst_v, *rest_refs):
        bufs = rest_refs[:_D]
        acc_sh = rest_refs[_D]
        sems = rest_refs[_D + 1:]
        c = lax.axis_index("c")
        s = lax.axis_index("s")
        start = jnp.where(
            c == 0,
            s * nc0,
            nc0 * _NS + _D * (s * p1 + jnp.minimum(s, extra)))
        my_nc = jnp.where(c == 0, nc0,
                          jnp.where(s < extra, nc1 + _D, nc1))

        @pl.when(c == 0)
        def _():
            pltpu.sync_copy(edges_hbm.at[0, pl.ds(start, nc0)],
                            src_v.at[pl.ds(0, nc0)])
            pltpu.sync_copy(edges_hbm.at[1, pl.ds(start, nc0)],
                            dst_v.at[pl.ds(0, nc0)])

        @pl.when((c == 1) & (s < extra))
        def _():
            pltpu.sync_copy(edges_hbm.at[0, pl.ds(start, nc1 + _D)],
                            src_v.at[pl.ds(0, nc1 + _D)])
            pltpu.sync_copy(edges_hbm.at[1, pl.ds(start, nc1 + _D)],
                            dst_v.at[pl.ds(0, nc1 + _D)])

        @pl.when((c == 1) & (s >= extra))
        def _():
            pltpu.sync_copy(edges_hbm.at[0, pl.ds(start, nc1)],
                            src_v.at[pl.ds(0, nc1)])
            pltpu.sync_copy(edges_hbm.at[1, pl.ds(start, nc1)],
                            dst_v.at[pl.ds(0, nc1)])

        pltpu.sync_copy(zeros_hbm, acc_sh.at[pl.ds(s * zr, zr)])
        plsc.subcore_barrier()

        def body(q, carry):
            j = _D * q
            for k in range(_D):
                jn = lax.rem(j + _D - 1 + k, my_nc)
                pltpu.async_copy(h_hbm.at[src_v.at[jn]],
                                 bufs[(k + _D - 1) % _D],
                                 sems[(k + _D - 1) % _D])
                pltpu.make_async_copy(h_hbm.at[src_v.at[j + k]],
                                      bufs[k], sems[k]).wait()
                pltpu.sync_copy(bufs[k], acc_sh.at[dst_v.at[j + k]], add=True)
            return carry

        for k in range(_D - 1):
            pltpu.async_copy(h_hbm.at[src_v.at[k]], bufs[k], sems[k])
        lax.fori_loop(0, my_nc // _D, body, 0)
        for k in range(_D - 1):
            pltpu.make_async_copy(h_hbm.at[src_v.at[k]], bufs[k], sems[k]).wait()
        plsc.subcore_barrier()
        pltpu.sync_copy(acc_sh.at[pl.ds(s * zr, zr)],
                        out_hbm.at[pl.ds(s * zr, zr), pl.ds(c * d_h, d_h)])

    return spmm(h, edges3, zeros_blk)


def _epilogue(h, parts, skip, bg, wp, bp):
    n, d_h = h.shape
    k = wp.shape[1]
    rb = 2000 if n % 2000 == 0 else 8
    grid = n // rb

    def body(h_ref, p_ref, skip_ref, bg_ref, wp_ref, bp_ref,
             asg_ref, pool_ref, acc_ref):
        i = pl.program_id(0)
        agg = p_ref[:, :d_h] + p_ref[:, d_h:]
        h2 = _selu(skip_ref[:] * h_ref[:] + agg + bg_ref[:])
        logits = jnp.dot(h2, wp_ref[:], preferred_element_type=jnp.float32)
        logits = logits + bp_ref[:]
        m = jnp.max(logits, axis=-1, keepdims=True)
        e = jnp.exp(logits - m)
        a = e / jnp.sum(e, axis=-1, keepdims=True)
        asg_ref[:] = a
        @pl.when(i == 0)
        def _():
            acc_ref[:] = jnp.zeros_like(acc_ref)
        acc_ref[:] += lax.dot_general(
            a, h2, (((0,), (0,)), ((), ())), preferred_element_type=jnp.float32)
        @pl.when(i == pl.num_programs(0) - 1)
        def _():
            pool_ref[:] = _selu(acc_ref[:])

    asg, pool = pl.pallas_call(
        body,
        grid=(grid,),
        in_specs=[
            pl.BlockSpec((rb, d_h), lambda i: (i, 0)),
            pl.BlockSpec((rb, _NC * d_h), lambda i: (i, 0)),
            pl.BlockSpec((1, d_h), lambda i: (0, 0)),
            pl.BlockSpec((1, d_h), lambda i: (0, 0)),
            pl.BlockSpec((d_h, k), lambda i: (0, 0)),
            pl.BlockSpec((1, k), lambda i: (0, 0)),
        ],
        out_specs=[
            pl.BlockSpec((rb, k), lambda i: (i, 0)),
            pl.BlockSpec((k, d_h), lambda i: (0, 0)),
        ],
        out_shape=[
            jax.ShapeDtypeStruct((n, k), jnp.float32),
            jax.ShapeDtypeStruct((k, d_h), jnp.float32),
        ],
        scratch_shapes=[pltpu.VMEM((k, d_h), jnp.float32)],
    )(h, parts, skip, bg, wp, bp)
    return pool, asg


def kernel(features, edge_index, W_gcn, b_gcn, skip_gcn, W_pool, b_pool):
    n, _ = features.shape
    d_h = W_gcn.shape[1]
    e = edge_index.shape[1]

    h = _matmul(features, W_gcn)

    acc_rows = -(-(n + 1) // (_NS * 8)) * (_NS * 8)
    n_chunks = _D * (-(-e // (_CH * _D)))
    if e == n_chunks * _CH:
        edges3 = edge_index.reshape(2, n_chunks, _CH)
    else:
        pad = n_chunks * _CH - e
        edges3 = jnp.concatenate(
            [edge_index,
             jnp.stack([jnp.zeros((pad,), jnp.int32),
                        jnp.full((pad,), n, jnp.int32)])], axis=1,
        ).reshape(2, n_chunks, _CH)
    zeros_blk = jnp.zeros((acc_rows // _NS, d_h), jnp.float32)

    parts = _spmm_sc(h, edges3, zeros_blk, acc_rows, n_chunks)

    pool, asg = _epilogue(
        h, parts,
        skip_gcn.reshape(1, d_h), b_gcn.reshape(1, d_h),
        W_pool, b_pool.reshape(1, -1))
    return (pool, asg)


# speedup vs baseline: 1.1320x; 1.1320x over previous
import functools

import jax
import jax.numpy as jnp
from jax import lax
from jax.experimental import pallas as pl
from jax.experimental.pallas import tpu as pltpu
from jax.experimental.pallas import tpu_sc as plsc

_SELU_SCALE = 1.0507009873554805
_SELU_ALPHA = 1.6732632423543772

_NC = 2
_NS = 16
_CH = 128
_D = 4
_CORE0_SHARE = 0.5


def _selu(x):
    return _SELU_SCALE * jnp.where(x > 0, x, _SELU_ALPHA * (jnp.exp(x) - 1.0))


def _matmul(x, w):
    n, d_in = x.shape
    d_out = w.shape[1]
    rb = 2000 if n % 2000 == 0 else 8
    grid = n // rb

    def body(x_ref, w_ref, o_ref):
        o_ref[:] = jnp.dot(x_ref[:], w_ref[:], preferred_element_type=jnp.float32)

    return pl.pallas_call(
        body,
        grid=(grid,),
        in_specs=[
            pl.BlockSpec((rb, d_in), lambda i: (i, 0)),
            pl.BlockSpec((d_in, d_out), lambda i: (0, 0)),
        ],
        out_specs=pl.BlockSpec((rb, d_out), lambda i: (i, 0)),
        out_shape=jax.ShapeDtypeStruct((n, d_out), jnp.float32),
    )(x, w)


def _spmm_sc(h, edges3, zeros_blk, acc_rows, n_chunks):
    n, d_h = h.shape
    zr = acc_rows // _NS

    n_groups = n_chunks // _D
    p0 = max(1, min(n_groups // _NS - 1, round(n_groups * _CORE0_SHARE / _NS)))
    rest = n_groups - p0 * _NS
    p1 = rest // _NS
    extra = rest - p1 * _NS
    nc0 = _D * p0
    nc1 = _D * p1
    nc_max = _D * max(p0, p1 + (1 if extra else 0))

    mesh = plsc.VectorSubcoreMesh(
        core_axis_name="c", subcore_axis_name="s",
        num_cores=_NC, num_subcores=_NS)

    @functools.partial(
        pl.kernel,
        out_type=jax.ShapeDtypeStruct((acc_rows, _NC * d_h), jnp.float32),
        mesh=mesh,
        scratch_types=[
            pltpu.VMEM((nc_max, _CH), jnp.int32),
            pltpu.VMEM((nc_max, _CH), jnp.int32),
        ] + [pltpu.VMEM((_CH, d_h), jnp.float32)] * _D + [
            pltpu.VMEM_SHARED((acc_rows, d_h), jnp.float32),
        ] + [pltpu.SemaphoreType.DMA] * _D,
        compiler_params=pltpu.CompilerParams(use_tc_tiling_on_sc=False),
    )
    def spmm(h_hbm, edges_hbm, zeros_hbm, out_hbm,
             src_v, dst_v, *rest_refs):
        bufs = rest_refs[:_D]
        acc_sh = rest_refs[_D]
        sems = rest_refs[_D + 1:]
        c = lax.axis_index("c")
        s = lax.axis_index("s")
        start = jnp.where(
            c == 0,
            s * nc0,
            nc0 * _NS + _D * (s * p1 + jnp.minimum(s, extra)))
        my_nc = jnp.where(c == 0, nc0,
                          jnp.where(s < extra, nc1 + _D, nc1))

        @pl.when(c == 0)
        def _():
            pltpu.sync_copy(edges_hbm.at[0, pl.ds(start, nc0)],
                            src_v.at[pl.ds(0, nc0)])
            pltpu.sync_copy(edges_hbm.at[1, pl.ds(start, nc0)],
                            dst_v.at[pl.ds(0, nc0)])

        @pl.when((c == 1) & (s < extra))
        def _():
            pltpu.sync_copy(edges_hbm.at[0, pl.ds(start, nc1 + _D)],
                            src_v.at[pl.ds(0, nc1 + _D)])
            pltpu.sync_copy(edges_hbm.at[1, pl.ds(start, nc1 + _D)],
                            dst_v.at[pl.ds(0, nc1 + _D)])

        @pl.when((c == 1) & (s >= extra))
        def _():
            pltpu.sync_copy(edges_hbm.at[0, pl.ds(start, nc1)],
                            src_v.at[pl.ds(0, nc1)])
            pltpu.sync_copy(edges_hbm.at[1, pl.ds(start, nc1)],
                            dst_v.at[pl.ds(0, nc1)])

        pltpu.sync_copy(zeros_hbm, acc_sh.at[pl.ds(s * zr, zr)])
        plsc.subcore_barrier()

        def body(q, carry):
            j = _D * q
            for k in range(_D):
                jn = lax.rem(j + _D - 1 + k, my_nc)
                pltpu.async_copy(h_hbm.at[src_v.at[jn]],
                                 bufs[(k + _D - 1) % _D],
                                 sems[(k + _D - 1) % _D])
                pltpu.make_async_copy(h_hbm.at[src_v.at[j + k]],
                                      bufs[k], sems[k]).wait()
                pltpu.sync_copy(bufs[k], acc_sh.at[dst_v.at[j + k]], add=True)
            return carry

        for k in range(_D - 1):
            pltpu.async_copy(h_hbm.at[src_v.at[k]], bufs[k], sems[k])
        lax.fori_loop(0, my_nc // _D, body, 0)
        for k in range(_D - 1):
            pltpu.make_async_copy(h_hbm.at[src_v.at[k]], bufs[k], sems[k]).wait()
        plsc.subcore_barrier()
        pltpu.sync_copy(acc_sh.at[pl.ds(s * zr, zr)],
                        out_hbm.at[pl.ds(s * zr, zr), pl.ds(c * d_h, d_h)])

    return spmm(h, edges3, zeros_blk)


def _epilogue(h, parts, skip, bg, wp, bp):
    n, d_h = h.shape
    k = wp.shape[1]
    rb = 2000 if n % 2000 == 0 else 8
    grid = n // rb

    def body(h_ref, p_ref, skip_ref, bg_ref, wp_ref, bp_ref,
             asg_ref, pool_ref, acc_ref):
        i = pl.program_id(0)
        agg = p_ref[:, :d_h] + p_ref[:, d_h:]
        h2 = _selu(skip_ref[:] * h_ref[:] + agg + bg_ref[:])
        logits = jnp.dot(h2, wp_ref[:], preferred_element_type=jnp.float32)
        logits = logits + bp_ref[:]
        m = jnp.max(logits, axis=-1, keepdims=True)
        e = jnp.exp(logits - m)
        a = e / jnp.sum(e, axis=-1, keepdims=True)
        asg_ref[:] = a
        @pl.when(i == 0)
        def _():
            acc_ref[:] = jnp.zeros_like(acc_ref)
        acc_ref[:] += lax.dot_general(
            a, h2, (((0,), (0,)), ((), ())), preferred_element_type=jnp.float32)
        @pl.when(i == pl.num_programs(0) - 1)
        def _():
            pool_ref[:] = _selu(acc_ref[:])

    asg, pool = pl.pallas_call(
        body,
        grid=(grid,),
        in_specs=[
            pl.BlockSpec((rb, d_h), lambda i: (i, 0)),
            pl.BlockSpec((rb, _NC * d_h), lambda i: (i, 0)),
            pl.BlockSpec((1, d_h), lambda i: (0, 0)),
            pl.BlockSpec((1, d_h), lambda i: (0, 0)),
            pl.BlockSpec((d_h, k), lambda i: (0, 0)),
            pl.BlockSpec((1, k), lambda i: (0, 0)),
        ],
        out_specs=[
            pl.BlockSpec((rb, k), lambda i: (i, 0)),
            pl.BlockSpec((k, d_h), lambda i: (0, 0)),
        ],
        out_shape=[
            jax.ShapeDtypeStruct((n, k), jnp.float32),
            jax.ShapeDtypeStruct((k, d_h), jnp.float32),
        ],
        scratch_shapes=[pltpu.VMEM((k, d_h), jnp.float32)],
    )(h, parts, skip, bg, wp, bp)
    return pool, asg


def kernel(features, edge_index, W_gcn, b_gcn, skip_gcn, W_pool, b_pool):
    n, _ = features.shape
    d_h = W_gcn.shape[1]
    e = edge_index.shape[1]

    h = _matmul(features, W_gcn)

    acc_rows = -(-(n + 1) // (_NS * 8)) * (_NS * 8)
    n_chunks = _D * (-(-e // (_CH * _D)))
    if e == n_chunks * _CH:
        edges3 = edge_index.reshape(2, n_chunks, _CH)
    else:
        pad = n_chunks * _CH - e
        edges3 = jnp.concatenate(
            [edge_index,
             jnp.stack([jnp.zeros((pad,), jnp.int32),
                        jnp.full((pad,), n, jnp.int32)])], axis=1,
        ).reshape(2, n_chunks, _CH)
    zeros_blk = jnp.zeros((acc_rows // _NS, d_h), jnp.float32)

    parts = _spmm_sc(h, edges3, zeros_blk, acc_rows, n_chunks)

    pool, asg = _epilogue(
        h, parts,
        skip_gcn.reshape(1, d_h), b_gcn.reshape(1, d_h),
        W_pool, b_pool.reshape(1, -1))
    return (pool, asg)
